# baseline (speedup 1.0000x reference)
import jax
import jax.numpy as jnp
from jax.experimental import pallas as pl
from jax.experimental.pallas import tpu as pltpu

_ROWS, _COLS = 16384, 4096
_CH = 256
_NBUF = 4
_NCHUNKS = _ROWS // _CH
_NGROUPS = _NCHUNKS // _NBUF


def _ring_copy_patch(x_hbm, o_hbm, buf, in_sems, out_sems):
    def start_in(b, ci):
        pltpu.make_async_copy(
            x_hbm.at[pl.ds(ci * _CH, _CH), :], buf.at[b], in_sems.at[b]
        ).start()

    def wait_in(b):
        pltpu.make_async_copy(
            x_hbm.at[pl.ds(0, _CH), :], buf.at[b], in_sems.at[b]
        ).wait()

    def start_out(b, ci):
        pltpu.make_async_copy(
            buf.at[b], o_hbm.at[pl.ds(ci * _CH, _CH), :], out_sems.at[b]
        ).start()

    def wait_out(b):
        pltpu.make_async_copy(
            buf.at[b], o_hbm.at[pl.ds(0, _CH), :], out_sems.at[b]
        ).wait()

    for b in range(_NBUF):
        start_in(b, b)

    def group(g, _):
        for b in range(_NBUF):
            wait_in(b)

            if b == 0:
                @pl.when(g == 0)
                def _patch():
                    tile = buf[0, 0:8, 0:128]
                    r = jax.lax.broadcasted_iota(jnp.int32, (8, 128), 0)
                    c = jax.lax.broadcasted_iota(jnp.int32, (8, 128), 1)
                    tile = jnp.where((r == 0) & (c == 0), 10.0, tile)
                    tile = jnp.where((r == 0) & (c == 2), 30.0, tile)
                    tile = jnp.where((r == 1) & (c == 0), 40.0, tile)
                    tile = jnp.where((r == 1) & (c == 1), 20.0, tile)
                    buf[0, 0:8, 0:128] = tile

            start_out(b, g * _NBUF + b)
        for b in range(_NBUF):
            @pl.when(g + 1 < _NGROUPS)
            def _prefetch():
                wait_out(b)
                start_in(b, (g + 1) * _NBUF + b)
        return 0

    jax.lax.fori_loop(0, _NGROUPS, group, 0)
    for b in range(_NBUF):
        wait_out(b)


def kernel(x):
    return pl.pallas_call(
        _ring_copy_patch,
        in_specs=[pl.BlockSpec(memory_space=pl.ANY)],
        out_specs=pl.BlockSpec(memory_space=pl.ANY),
        out_shape=jax.ShapeDtypeStruct((_ROWS, _COLS), jnp.float32),
        scratch_shapes=[
            pltpu.VMEM((_NBUF, _CH, _COLS), jnp.float32),
            pltpu.SemaphoreType.DMA((_NBUF,)),
            pltpu.SemaphoreType.DMA((_NBUF,)),
        ],
    )(x)
